# Initial kernel scaffold; baseline (speedup 1.0000x reference)
import jax
import jax.numpy as jnp
from jax.experimental import pallas as pl


def kernel(origin_density, origin_mask, pre_density, new_mask):
    raise NotImplementedError("write your pallas kernel here")


# speedup vs baseline: 59.4683x; 59.4683x over previous
import functools

import jax
import jax.numpy as jnp
from jax import lax
from jax.experimental import pallas as pl
from jax.experimental.pallas import tpu as pltpu
from jax.experimental.pallas import tpu_sc as plsc

H = W = 2048
N = H * W
NUM_BINS = 64
NC, NS, L = 2, 16, 16
NW = NC * NS
PER_W = N // NW
CHUNK = 16384
NCHUNKS = PER_W // CHUNK
UNROLL = 8


def _sc_body(mask_hbm, od_hbm, pd_hbm, out_hbm,
             m0, m1, o0, o1, p0, p1, bins, row, sem0, sem1):
    c = lax.axis_index("c")
    s = lax.axis_index("s")
    wid = s * NC + c
    base = wid * PER_W

    zero16 = jnp.zeros((L,), jnp.float32)
    for r in range(L * NUM_BINS // L):
        bins[pl.ds(r * L, L)] = zero16

    bufs = ((m0, o0, p0, sem0), (m1, o1, p1, sem1))

    def start(g):
        mb, ob, pb, sem = bufs[g & 1]
        off = base + g * CHUNK
        return (pltpu.async_copy(mask_hbm.at[pl.ds(off, CHUNK)], mb, sem),
                pltpu.async_copy(od_hbm.at[pl.ds(off, CHUNK)], ob, sem),
                pltpu.async_copy(pd_hbm.at[pl.ds(off, CHUNK)], pb, sem))

    iota64 = lax.iota(jnp.int32, L) * NUM_BINS
    handles = {0: start(0)}
    for g in range(NCHUNKS):
        if g + 1 < NCHUNKS:
            handles[g + 1] = start(g + 1)
        for h in handles.pop(g):
            h.wait()
        mb, ob, pb, _ = bufs[g & 1]

        def cbody(i, carry, mb=mb, ob=ob, pb=pb):
            off = i * (L * UNROLL)
            for u in range(UNROLL):
                o = off + u * L
                m = mb[pl.ds(o, L)]
                d = pb[pl.ds(o, L)] - ob[pl.ds(o, L)]
                plsc.addupdate_scatter(bins, [iota64 + m], d)
            return carry

        lax.fori_loop(0, CHUNK // (L * UNROLL), cbody, 0)

    for j in range(NUM_BINS // L):
        acc = bins[pl.ds(j * L, L)]
        for r in range(1, L):
            acc = acc + bins[pl.ds(r * NUM_BINS + j * L, L)]
        row[pl.ds(j * L, L)] = acc
    pltpu.sync_copy(row, out_hbm.at[wid])


_sc_bins = functools.partial(
    pl.kernel,
    out_type=jax.ShapeDtypeStruct((NW, NUM_BINS), jnp.float32),
    mesh=plsc.VectorSubcoreMesh(core_axis_name="c", subcore_axis_name="s"),
    compiler_params=pltpu.CompilerParams(needs_layout_passes=False),
    scratch_types=[
        pltpu.VMEM((CHUNK,), jnp.int32),
        pltpu.VMEM((CHUNK,), jnp.int32),
        pltpu.VMEM((CHUNK,), jnp.float32),
        pltpu.VMEM((CHUNK,), jnp.float32),
        pltpu.VMEM((CHUNK,), jnp.float32),
        pltpu.VMEM((CHUNK,), jnp.float32),
        pltpu.VMEM((L * NUM_BINS,), jnp.float32),
        pltpu.VMEM((NUM_BINS,), jnp.float32),
        pltpu.SemaphoreType.DMA,
        pltpu.SemaphoreType.DMA,
    ],
)(_sc_body)


def _finish_body(p_ref, o_ref):
    b = jnp.sum(p_ref[...], axis=0)
    o_ref[0, 0] = jnp.sum(jnp.abs(b))


def _finish(partial):
    out = pl.pallas_call(
        _finish_body,
        out_shape=jax.ShapeDtypeStruct((1, 1), jnp.float32),
        in_specs=[pl.BlockSpec(memory_space=pltpu.VMEM)],
        out_specs=pl.BlockSpec(memory_space=pltpu.SMEM),
    )(partial)
    return out[0, 0]


def kernel(origin_density, origin_mask, pre_density, new_mask):
    del new_mask
    mask = origin_mask.reshape(-1).astype(jnp.int32)
    od = origin_density.reshape(-1)
    pd = pre_density.reshape(-1)
    partial = _sc_bins(mask, od, pd)
    return _finish(partial)


# speedup vs baseline: 155.8439x; 2.6206x over previous
import functools

import jax
import jax.numpy as jnp
from jax import lax
from jax.experimental import pallas as pl
from jax.experimental.pallas import tpu as pltpu
from jax.experimental.pallas import tpu_sc as plsc

H = W = 2048
NUM_BINS = 64
NC, NS, L = 2, 16, 16
NW = NC * NS
ROWS_PER_W = H // NW
ROW_CHUNK = 8
NCHUNKS = ROWS_PER_W // ROW_CHUNK
VECS_PER_ROW = W // L


def _sc_body(od_hbm, mask_hbm, pd_hbm, out_hbm,
             m0, m1, o0, o1, p0, p1, bins, row, sem0, sem1):
    c = lax.axis_index("c")
    s = lax.axis_index("s")
    wid = s * NC + c
    base = wid * ROWS_PER_W

    zero16 = jnp.zeros((L,), jnp.float32)
    for r in range(L * NUM_BINS // L):
        bins[pl.ds(r * L, L)] = zero16

    bufs = ((m0, o0, p0, sem0), (m1, o1, p1, sem1))

    def start(g):
        mb, ob, pb, sem = bufs[g & 1]
        r0 = base + g * ROW_CHUNK
        return (pltpu.async_copy(mask_hbm.at[pl.ds(r0, ROW_CHUNK)], mb, sem),
                pltpu.async_copy(od_hbm.at[pl.ds(r0, ROW_CHUNK)], ob, sem),
                pltpu.async_copy(pd_hbm.at[pl.ds(r0, ROW_CHUNK)], pb, sem))

    iota64 = lax.iota(jnp.int32, L) * NUM_BINS
    handles = {0: start(0)}
    for g in range(NCHUNKS):
        if g + 1 < NCHUNKS:
            handles[g + 1] = start(g + 1)
        for h in handles.pop(g):
            h.wait()
        mb, ob, pb, _ = bufs[g & 1]

        def cbody(i, carry, mb=mb, ob=ob, pb=pb):
            o = i * L
            ms = [mb[r, pl.ds(o, L)] for r in range(ROW_CHUNK)]
            ps = [pb[r, pl.ds(o, L)] for r in range(ROW_CHUNK)]
            osv = [ob[r, pl.ds(o, L)] for r in range(ROW_CHUNK)]
            for r in range(ROW_CHUNK):
                plsc.addupdate_scatter(bins, [iota64 + ms[r]], ps[r] - osv[r])
            return carry

        lax.fori_loop(0, VECS_PER_ROW, cbody, 0)

    for j in range(NUM_BINS // L):
        acc = bins[pl.ds(j * L, L)]
        for r in range(1, L):
            acc = acc + bins[pl.ds(r * NUM_BINS + j * L, L)]
        row[pl.ds(j * L, L)] = acc
    pltpu.sync_copy(row, out_hbm.at[wid])


_sc_bins = functools.partial(
    pl.kernel,
    out_type=jax.ShapeDtypeStruct((NW, NUM_BINS), jnp.float32),
    mesh=plsc.VectorSubcoreMesh(core_axis_name="c", subcore_axis_name="s"),
    compiler_params=pltpu.CompilerParams(needs_layout_passes=False),
    scratch_types=[
        pltpu.VMEM((ROW_CHUNK, W), jnp.int32),
        pltpu.VMEM((ROW_CHUNK, W), jnp.int32),
        pltpu.VMEM((ROW_CHUNK, W), jnp.float32),
        pltpu.VMEM((ROW_CHUNK, W), jnp.float32),
        pltpu.VMEM((ROW_CHUNK, W), jnp.float32),
        pltpu.VMEM((ROW_CHUNK, W), jnp.float32),
        pltpu.VMEM((L * NUM_BINS,), jnp.float32),
        pltpu.VMEM((NUM_BINS,), jnp.float32),
        pltpu.SemaphoreType.DMA,
        pltpu.SemaphoreType.DMA,
    ],
)(_sc_body)


def _finish_body(p_ref, o_ref):
    b = jnp.sum(p_ref[...], axis=0)
    o_ref[0, 0] = jnp.sum(jnp.abs(b))


def _finish(partial):
    out = pl.pallas_call(
        _finish_body,
        out_shape=jax.ShapeDtypeStruct((1, 1), jnp.float32),
        in_specs=[pl.BlockSpec(memory_space=pltpu.VMEM)],
        out_specs=pl.BlockSpec(memory_space=pltpu.SMEM),
    )(partial)
    return out[0, 0]


def kernel(origin_density, origin_mask, pre_density, new_mask):
    del new_mask
    mask = origin_mask.astype(jnp.int32)
    partial = _sc_bins(origin_density, mask, pre_density)
    return _finish(partial)


# speedup vs baseline: 156.7899x; 1.0061x over previous
import functools

import jax
import jax.numpy as jnp
from jax import lax
from jax.experimental import pallas as pl
from jax.experimental.pallas import tpu as pltpu
from jax.experimental.pallas import tpu_sc as plsc

H = W = 2048
NUM_BINS = 64
NC, NS, L = 2, 16, 16
NW = NC * NS
ROWS_PER_W = H // NW
ROW_CHUNK = 8
NCHUNKS = ROWS_PER_W // ROW_CHUNK
VECS_PER_ROW = W // L


def _sc_body(od_hbm, mask_hbm, pd_hbm, out_hbm,
             m0, m1, o0, o1, p0, p1, bins, row, sem0, sem1):
    c = lax.axis_index("c")
    s = lax.axis_index("s")
    wid = s * NC + c
    base = wid * ROWS_PER_W

    zero16 = jnp.zeros((L,), jnp.float32)
    for r in range(L * NUM_BINS // L):
        bins[pl.ds(r * L, L)] = zero16

    bufs = ((m0, o0, p0, sem0), (m1, o1, p1, sem1))

    def start(g):
        mb, ob, pb, sem = bufs[g & 1]
        r0 = base + g * ROW_CHUNK
        return (pltpu.async_copy(mask_hbm.at[pl.ds(r0, ROW_CHUNK)], mb, sem),
                pltpu.async_copy(od_hbm.at[pl.ds(r0, ROW_CHUNK)], ob, sem),
                pltpu.async_copy(pd_hbm.at[pl.ds(r0, ROW_CHUNK)], pb, sem))

    iota64 = lax.iota(jnp.int32, L) * NUM_BINS
    handles = {0: start(0)}
    for g in range(NCHUNKS):
        if g + 1 < NCHUNKS:
            handles[g + 1] = start(g + 1)
        for h in handles.pop(g):
            h.wait()
        mb, ob, pb, _ = bufs[g & 1]

        def cbody(i, carry, mb=mb, ob=ob, pb=pb):
            o = i * L
            ms = [mb[r, pl.ds(o, L)] for r in range(ROW_CHUNK)]
            ps = [pb[r, pl.ds(o, L)] for r in range(ROW_CHUNK)]
            osv = [ob[r, pl.ds(o, L)] for r in range(ROW_CHUNK)]
            for r in range(ROW_CHUNK):
                plsc.addupdate_scatter(bins, [iota64 + ms[r]], ps[r] - osv[r])
            return carry + ps[0]

        acc = plsc.parallel_loop(
            0, VECS_PER_ROW, unroll=2,
            carry=jnp.zeros((L,), jnp.float32))(cbody)
        plsc.addupdate(bins.at[pl.ds(0, L)], acc * 0.0)

    for j in range(NUM_BINS // L):
        acc = bins[pl.ds(j * L, L)]
        for r in range(1, L):
            acc = acc + bins[pl.ds(r * NUM_BINS + j * L, L)]
        row[pl.ds(j * L, L)] = acc
    pltpu.sync_copy(row, out_hbm.at[wid])


_sc_bins = functools.partial(
    pl.kernel,
    out_type=jax.ShapeDtypeStruct((NW, NUM_BINS), jnp.float32),
    mesh=plsc.VectorSubcoreMesh(core_axis_name="c", subcore_axis_name="s"),
    compiler_params=pltpu.CompilerParams(needs_layout_passes=False),
    scratch_types=[
        pltpu.VMEM((ROW_CHUNK, W), jnp.int32),
        pltpu.VMEM((ROW_CHUNK, W), jnp.int32),
        pltpu.VMEM((ROW_CHUNK, W), jnp.float32),
        pltpu.VMEM((ROW_CHUNK, W), jnp.float32),
        pltpu.VMEM((ROW_CHUNK, W), jnp.float32),
        pltpu.VMEM((ROW_CHUNK, W), jnp.float32),
        pltpu.VMEM((L * NUM_BINS,), jnp.float32),
        pltpu.VMEM((NUM_BINS,), jnp.float32),
        pltpu.SemaphoreType.DMA,
        pltpu.SemaphoreType.DMA,
    ],
)(_sc_body)


def _finish_body(p_ref, o_ref):
    b = jnp.sum(p_ref[...], axis=0)
    o_ref[0, 0] = jnp.sum(jnp.abs(b))


def _finish(partial):
    out = pl.pallas_call(
        _finish_body,
        out_shape=jax.ShapeDtypeStruct((1, 1), jnp.float32),
        in_specs=[pl.BlockSpec(memory_space=pltpu.VMEM)],
        out_specs=pl.BlockSpec(memory_space=pltpu.SMEM),
    )(partial)
    return out[0, 0]


def kernel(origin_density, origin_mask, pre_density, new_mask):
    del new_mask
    mask = origin_mask.astype(jnp.int32)
    partial = _sc_bins(origin_density, mask, pre_density)
    return _finish(partial)
